# baseline (speedup 1.0000x reference)
import functools

import jax
import jax.numpy as jnp
from jax import lax
from jax.experimental import pallas as pl
from jax.experimental.pallas import tpu as pltpu
from jax.experimental.pallas import tpu_sc as plsc

B = 16384
EMB = 64
ITEM_V = 1000000
PAIR_H = 491520
PAIR_R = 508480
PAIR_SPLIT = PAIR_R
TBLK = 16384
NC = 2
NS = 16
NW = NC * NS
BPW = B // NW
CH = 128
NCH = BPW // CH
IR = B // CH
NBND = 1024


def _tc_transpose_body(l_ref, r_ref, out_ref):
    out_ref[...] = jnp.concatenate([l_ref[...].T, r_ref[...].T], axis=1)


def _tc_transpose(item_tt):
    grid = (PAIR_R + TBLK - 1) // TBLK
    return pl.pallas_call(
        _tc_transpose_body,
        grid=(grid,),
        in_specs=[
            pl.BlockSpec((EMB, TBLK), lambda i: (0, i)),
            pl.BlockSpec((EMB, TBLK), lambda i: (0, PAIR_H // TBLK + i)),
        ],
        out_specs=pl.BlockSpec((TBLK, 2 * EMB), lambda i: (i, 0)),
        out_shape=jax.ShapeDtypeStruct((PAIR_R, 2 * EMB), jnp.float32),
    )(item_tt, item_tt)


def _sc_item_body(item_i, item_t, item_o, idx_v, pair_a, pair_b, sem_a, sem_b):
    wid = lax.axis_index("s") * NC + lax.axis_index("c")
    rbase = wid * NCH
    obase = wid * BPW
    pltpu.sync_copy(item_i.at[pl.ds(rbase, NCH)], idx_v)
    bufs = [pair_a, pair_b]
    sems = [sem_a, sem_b]
    copies = [None, None]

    def fire(j):
        copies[j % 2] = pltpu.async_copy(
            item_t.at[idx_v.at[j]], bufs[j % 2], sems[j % 2])

    fire(0)
    fire(1)
    for j in range(NCH):
        copies[j % 2].wait()
        pltpu.sync_copy(bufs[j % 2], item_o.at[pl.ds(obase + j * CH, CH)])
        if j + 2 < NCH:
            fire(j + 2)


def _sc_body(bus_i, typ_i, sub_i, price_h, bnd_h,
             bus_t, typ_t, sub_t, price_t,
             bus_o, typ_o, sub_o, price_o,
             idx_v, bins_v, price_v, bnd_v, tab_v, out_v, sem):
    wid = lax.axis_index("s") * NC + lax.axis_index("c")
    base = wid * BPW

    pltpu.sync_copy(bus_i.at[pl.ds(base, BPW)], idx_v.at[pl.ds(0, BPW)])
    pltpu.sync_copy(typ_i.at[pl.ds(base, BPW)], idx_v.at[pl.ds(BPW, BPW)])
    pltpu.sync_copy(sub_i.at[pl.ds(base, BPW)], idx_v.at[pl.ds(2 * BPW, BPW)])
    pltpu.sync_copy(price_h.at[pl.ds(base, BPW)], price_v)
    pltpu.sync_copy(bnd_h, bnd_v)

    for g in range(BPW // 16):
        v = price_v[pl.ds(g * 16, 16)]
        pos = jnp.zeros((16,), jnp.int32)
        n = NBND
        while n > 1:
            half = n // 2
            probe = plsc.load_gather(bnd_v, [pos + (half - 1)])
            pos = pos + jnp.where(probe < v, half, 0)
            n -= half
        probe = plsc.load_gather(bnd_v, [pos])
        pos = pos + jnp.where(probe < v, 1, 0)
        bins_v[pl.ds(g * 16, 16)] = pos

    for t, tab in enumerate([bus_t, typ_t, sub_t, price_t]):
        pltpu.sync_copy(tab, tab_v)

        def group(g, _):
            if t < 3:
                cols = idx_v[pl.ds(t * BPW + g * 16, 16)]
            else:
                cols = bins_v[pl.ds(g * 16, 16)]
            for d in range(EMB):
                v = plsc.load_gather(tab_v, [jnp.full((16,), d, jnp.int32),
                                             cols])
                out_v[d, pl.ds(g * 16, 16)] = v
            return 0

        lax.fori_loop(0, BPW // 16, group, 0)
        out = [bus_o, typ_o, sub_o, price_o][t]
        pltpu.sync_copy(out_v, out.at[:, pl.ds(base, BPW)])


def _sc_item_gather(item_i, item_t):
    f = functools.partial(
        pl.kernel,
        out_type=jax.ShapeDtypeStruct((B, 2 * EMB), jnp.float32),
        mesh=plsc.VectorSubcoreMesh(core_axis_name="c", subcore_axis_name="s"),
        scratch_types=[
            pltpu.VMEM((NCH, CH), jnp.int32),
            pltpu.VMEM((CH, 2 * EMB), jnp.float32),
            pltpu.VMEM((CH, 2 * EMB), jnp.float32),
            pltpu.SemaphoreType.DMA,
            pltpu.SemaphoreType.DMA,
        ],
        compiler_params=pltpu.CompilerParams(needs_layout_passes=False,
                                             use_tc_tiling_on_sc=True),
        name="item_model_sc_item_gather",
    )(_sc_item_body)
    return f(item_i, item_t)


def _sc_gather(bus_i, typ_i, sub_i, price_i, bnd,
               bus_tt, typ_tt, sub_tt, price_tt):
    col = jax.ShapeDtypeStruct((EMB, B), jnp.float32)
    f = functools.partial(
        pl.kernel,
        out_type=[col] * 4,
        mesh=plsc.VectorSubcoreMesh(core_axis_name="c", subcore_axis_name="s"),
        scratch_types=[
            pltpu.VMEM((3 * BPW,), jnp.int32),
            pltpu.VMEM((BPW,), jnp.int32),
            pltpu.VMEM((BPW,), jnp.float32),
            pltpu.VMEM((NBND,), jnp.float32),
            pltpu.VMEM((EMB, 1001), jnp.float32),
            pltpu.VMEM((EMB, BPW), jnp.float32),
            pltpu.SemaphoreType.DMA,
        ],
        compiler_params=pltpu.CompilerParams(needs_layout_passes=False,
                                             use_tc_tiling_on_sc=True),
        name="item_model_sc_gather",
    )(_sc_body)
    return f(bus_i, typ_i, sub_i, price_i, bnd, bus_tt, typ_tt, sub_tt,
             price_tt)


def _tc_body(pair_r, par_r, bus_r, typ_r, sub_r, price_r, img_r,
             wc_r, bc_r, wd_r, bd_r, out_r):
    p = par_r[...]
    pair_t = pair_r[...].T
    item = pair_t[0:EMB, :] * (1.0 - p) + pair_t[EMB:2 * EMB, :] * p
    attrs = jnp.concatenate([bus_r[...], typ_r[...], sub_r[...]], axis=0)
    u = jax.lax.dot_general(wc_r[...], attrs, (((0,), (0,)), ((), ())),
                            preferred_element_type=jnp.float32) + bc_r[...]
    cross = attrs * u + attrs
    img = jax.lax.dot_general(wd_r[...], img_r[...], (((0,), (0,)), ((), ())),
                              preferred_element_type=jnp.float32)
    img = jnp.maximum(img + bd_r[...], 0.0)
    out_r[...] = jnp.concatenate([item, cross, price_r[...], img], axis=0)


def _tc_combine(pair_r, par, bus_c, typ_c, sub_c, price_c, img_t,
                cross_W, cross_b, dense_W, dense_b):
    blk = 1024
    grid = B // blk
    cols = pl.BlockSpec((EMB, blk), lambda i: (0, i))
    out_t = pl.pallas_call(
        _tc_body,
        grid=(grid,),
        in_specs=[
            pl.BlockSpec((blk, 2 * EMB), lambda i: (i, 0)),
            pl.BlockSpec((1, blk), lambda i: (0, i)),
            cols, cols, cols, cols,
            pl.BlockSpec((12, blk), lambda i: (0, i)),
            pl.BlockSpec((3 * EMB, 3 * EMB), lambda i: (0, 0)),
            pl.BlockSpec((3 * EMB, 1), lambda i: (0, 0)),
            pl.BlockSpec((12, 12), lambda i: (0, 0)),
            pl.BlockSpec((12, 1), lambda i: (0, 0)),
        ],
        out_specs=pl.BlockSpec((332, blk), lambda i: (0, i)),
        out_shape=jax.ShapeDtypeStruct((332, B), jnp.float32),
    )(pair_r, par, bus_c, typ_c, sub_c, price_c, img_t,
      cross_W, cross_b, dense_W, dense_b)
    return out_t.T


def kernel(last_product_id, last_product_business_desc, last_product_type_desc,
           last_product_sub_category, last_product_list_price,
           last_image_embedding_pca, item_table, business_table, type_table,
           subcat_table, price_table, price_boundaries, cross_W, cross_b,
           dense_W, dense_b):
    right = last_product_id >= PAIR_SPLIT
    item_i = jnp.where(right, last_product_id - PAIR_H,
                       last_product_id).reshape(IR, CH)
    bnd = jnp.concatenate(
        [price_boundaries,
         jnp.full((NBND - price_boundaries.shape[0],), jnp.inf, jnp.float32)])
    item_tt = item_table.T
    par = right.astype(jnp.float32).reshape(1, B)
    bus_c, typ_c, sub_c, price_c = _sc_gather(
        last_product_business_desc, last_product_type_desc,
        last_product_sub_category, last_product_list_price, bnd,
        business_table.T, type_table.T, subcat_table.T, price_table.T)
    pair_table = _tc_transpose(item_tt)
    item_i, bus_c = jax.lax.optimization_barrier((item_i, bus_c))
    pair_r = _sc_item_gather(item_i, pair_table)
    return _tc_combine(pair_r, par, bus_c, typ_c, sub_c, price_c,
                       last_image_embedding_pca.T, cross_W,
                       cross_b.reshape(3 * EMB, 1), dense_W,
                       dense_b.reshape(12, 1))
